# baseline (device time: 38473 ns/iter reference)
import jax
import jax.numpy as jnp
from jax import lax
from jax.experimental import pallas as pl
from jax.experimental.pallas import tpu as pltpu

N_LAYERS = 3


def kernel(x, Win0, Wout0, Win1, Wout1, Win2, Wout2):
    b, d_in = x.shape
    _, h_blk = Win0.shape

    def body(x_ref, win0_ref, wout0_ref, win1_ref, wout1_ref,
             win2_ref, wout2_ref, out_ref,
             acc_h, acc_x, recv_h, recv_x, send_sems, recv_sems):
        my_x = lax.axis_index("x")
        my_y = lax.axis_index("y")
        y_peer = (my_x, 1 - my_y)
        x_peer = (1 - my_x, my_y)

        barrier_sem = pltpu.get_barrier_semaphore()
        for nbr in (y_peer, x_peer):
            pl.semaphore_signal(
                barrier_sem, inc=1,
                device_id=nbr, device_id_type=pl.DeviceIdType.MESH,
            )
        pl.semaphore_wait(barrier_sem, 2)

        win_refs = (win0_ref, win1_ref, win2_ref)
        wout_refs = (wout0_ref, wout1_ref, wout2_ref)

        xcur = x_ref[:, :]
        for l in range(N_LAYERS):
            acc_h[:, :] = jnp.dot(
                xcur, win_refs[l][:, :], preferred_element_type=jnp.float32
            )
            rdma = pltpu.make_async_remote_copy(
                src_ref=acc_h,
                dst_ref=recv_h.at[l],
                send_sem=send_sems.at[2 * l],
                recv_sem=recv_sems.at[2 * l],
                device_id=y_peer,
                device_id_type=pl.DeviceIdType.MESH,
            )
            rdma.start()
            rdma.wait()
            h = jnp.maximum(acc_h[:, :] + recv_h[l, :, :], 0.0)

            acc_x[:, :] = jnp.dot(
                h, wout_refs[l][:, :], preferred_element_type=jnp.float32
            )
            rdma2 = pltpu.make_async_remote_copy(
                src_ref=acc_x,
                dst_ref=recv_x.at[l],
                send_sem=send_sems.at[2 * l + 1],
                recv_sem=recv_sems.at[2 * l + 1],
                device_id=x_peer,
                device_id_type=pl.DeviceIdType.MESH,
            )
            rdma2.start()
            rdma2.wait()
            xcur = acc_x[:, :] + recv_x[l, :, :]

        out_ref[:, :] = xcur

    return pl.pallas_call(
        body,
        out_shape=jax.ShapeDtypeStruct((b, d_in), jnp.float32),
        in_specs=[pl.BlockSpec(memory_space=pltpu.VMEM)] * 7,
        out_specs=pl.BlockSpec(memory_space=pltpu.VMEM),
        scratch_shapes=[
            pltpu.VMEM((b, h_blk), jnp.float32),
            pltpu.VMEM((b, d_in), jnp.float32),
            pltpu.VMEM((N_LAYERS, b, h_blk), jnp.float32),
            pltpu.VMEM((N_LAYERS, b, d_in), jnp.float32),
            pltpu.SemaphoreType.DMA((2 * N_LAYERS,)),
            pltpu.SemaphoreType.DMA((2 * N_LAYERS,)),
        ],
        compiler_params=pltpu.CompilerParams(collective_id=0),
    )(x, Win0, Wout0, Win1, Wout1, Win2, Wout2)
